# speedup vs baseline: 2.9385x; 2.9385x over previous
import jax
import jax.numpy as jnp
from jax.experimental import pallas as pl


def _mlp_head(pooled_ref, wf1_ref, bf1_ref, wf2_ref, bf2_ref, out_ref):
    h = jnp.dot(pooled_ref[...], wf1_ref[...], preferred_element_type=jnp.float32)
    h = jax.nn.relu(h + bf1_ref[...][None, :])
    o = jnp.dot(h, wf2_ref[...], preferred_element_type=jnp.float32)
    out_ref[...] = o + bf2_ref[...][None, :]


def kernel(x, edge_index, W1, b1, W2, b2, Wf1, bf1, Wf2, bf2):
    num_nodes = x.shape[0]
    ei = edge_index.astype(jnp.int32)
    src, dst = ei[0], ei[1]
    deg = jnp.ones((num_nodes,), jnp.float32).at[dst].add(1.0)
    dinv = jax.lax.rsqrt(deg)
    xw = x @ W1
    ts1 = xw * dinv[:, None]
    acc1 = ts1.at[dst].add(ts1[src])
    h1 = jax.nn.relu(dinv[:, None] * acc1 + b1)
    ts2 = h1 * dinv[:, None]
    acc2 = ts2.at[dst].add(ts2[src])
    h2 = jax.nn.relu((dinv[:, None] * acc2) @ W2 + b2)
    pooled = jnp.mean(h2, axis=0, keepdims=True)
    out = pl.pallas_call(
        _mlp_head,
        out_shape=jax.ShapeDtypeStruct((1, Wf2.shape[1]), jnp.float32),
    )(pooled, Wf1, bf1, Wf2, bf2)
    return out


# speedup vs baseline: 48.8821x; 16.6350x over previous
import functools

import jax
import jax.numpy as jnp
from jax import lax
from jax.experimental import pallas as pl
from jax.experimental.pallas import tpu as pltpu
from jax.experimental.pallas import tpu_sc as plsc

N_NODES = 10000
NR = 10240
NPT = NR // 16
F1 = 16
F2 = 32
G = 128
N_TILES = 16


def _mm_body(x_ref, w_ref, o_ref):
    o_ref[...] = jnp.dot(x_ref[...], w_ref[...],
                         preferred_element_type=jnp.float32)


def _rsqrt16(d):
    i = lax.bitcast_convert_type(d, jnp.int32)
    y = lax.bitcast_convert_type(jnp.int32(0x5F3759DF) - (i >> 1), jnp.float32)
    for _ in range(3):
        y = y * (1.5 - 0.5 * d * y * y)
    return y


def _sc_body(GT, xw_ref, src_ref, dst_ref, b1_ref, w2_ref, b2_ref,
             wf1_ref, bf1_ref, wf2_ref, bf2_ref, out_ref,
             table, acc, deg, pool,
             sidx, didx, rowbuf, nbin, nbout, dv, ones,
             wb1, wW2, wb2, wWf1, wbf1, wWf2, wbf2,
             poolb, poolall, outb):
    c = lax.axis_index("c")
    s = lax.axis_index("s")
    nbase = s * NPT
    gbase = s * GT

    pltpu.sync_copy(src_ref.at[pl.ds(gbase, GT)], sidx)
    pltpu.sync_copy(dst_ref.at[pl.ds(gbase, GT)], didx)
    pltpu.sync_copy(b1_ref, wb1)
    pltpu.sync_copy(w2_ref, wW2)
    pltpu.sync_copy(b2_ref, wb2)
    pltpu.sync_copy(wf1_ref, wWf1)
    pltpu.sync_copy(bf1_ref, wbf1)
    pltpu.sync_copy(wf2_ref, wWf2)
    pltpu.sync_copy(bf2_ref, wbf2)
    for j in range(G // 16):
        ones[pl.ds(j * 16, 16)] = jnp.full((16,), 1.0, jnp.float32)

    def initdeg(j, carry):
        dv[pl.ds(j * 16, 16)] = jnp.full((16,), 1.0, jnp.float32)
        return carry
    lax.fori_loop(0, NPT // 16, initdeg, 0)
    pltpu.sync_copy(dv, deg.at[pl.ds(nbase, NPT)])
    plsc.subcore_barrier()

    def dpass(g, carry):
        pltpu.sync_copy(ones, deg.at[didx.at[g]], add=True)
        return carry
    lax.fori_loop(0, GT, dpass, 0)
    plsc.subcore_barrier()

    pltpu.sync_copy(deg.at[pl.ds(nbase, NPT)], dv)

    def newton(j, carry):
        dv[pl.ds(j * 16, 16)] = _rsqrt16(dv[pl.ds(j * 16, 16)])
        return carry
    lax.fori_loop(0, NPT // 16, newton, 0)
    pltpu.sync_copy(xw_ref.at[pl.ds(nbase, NPT)], nbin)

    def scale1(j, carry):
        dvec = dv[pl.ds(j * 16, 16)]
        for l in range(16):
            i = j * 16 + l
            nbout[i, :] = nbin[i, :] * dvec[l]
        return carry
    lax.fori_loop(0, NPT // 16, scale1, 0)
    pltpu.sync_copy(nbout, table.at[pl.ds(nbase, NPT)])
    pltpu.sync_copy(nbout, acc.at[pl.ds(nbase, NPT)])
    plsc.subcore_barrier()

    def epass(g, carry):
        pltpu.sync_copy(table.at[sidx.at[g]], rowbuf)
        pltpu.sync_copy(rowbuf, acc.at[didx.at[g]], add=True)
        return carry

    lax.fori_loop(0, GT, epass, 0)
    plsc.subcore_barrier()

    pltpu.sync_copy(acc.at[pl.ds(nbase, NPT)], nbin)
    b1v = wb1[...]

    def stage_c(j, carry):
        dvec = dv[pl.ds(j * 16, 16)]
        for l in range(16):
            i = j * 16 + l
            d = dvec[l]
            h = jnp.maximum(nbin[i, :] * d + b1v, 0.0)
            nbout[i, :] = h * d
        return carry
    lax.fori_loop(0, NPT // 16, stage_c, 0)
    pltpu.sync_copy(nbout, table.at[pl.ds(nbase, NPT)])
    pltpu.sync_copy(nbout, acc.at[pl.ds(nbase, NPT)])
    plsc.subcore_barrier()

    lax.fori_loop(0, GT, epass, 0)
    plsc.subcore_barrier()

    pltpu.sync_copy(acc.at[pl.ds(nbase, NPT)], nbin)
    b2lo = wb2[pl.ds(0, 16)]
    b2hi = wb2[pl.ds(16, 16)]

    def stage_e(j, carry):
        plo, phi = carry
        dvec = dv[pl.ds(j * 16, 16)]
        for l in range(16):
            i = j * 16 + l
            aggv = nbin[i, :] * dvec[l]
            lo = b2lo
            hi = b2hi
            for k in range(F1):
                a = aggv[k]
                lo = lo + a * wW2[k, pl.ds(0, 16)]
                hi = hi + a * wW2[k, pl.ds(16, 16)]
            lo = jnp.maximum(lo, 0.0)
            hi = jnp.maximum(hi, 0.0)
            m = jnp.where(nbase + i < N_NODES, 1.0, 0.0)
            plo = plo + lo * m
            phi = phi + hi * m
        return (plo, phi)

    zero16 = jnp.zeros((16,), jnp.float32)
    plo, phi = lax.fori_loop(0, NPT // 16, stage_e, (zero16, zero16))
    poolb[0, pl.ds(0, 16)] = plo
    poolb[0, pl.ds(16, 16)] = phi
    pltpu.sync_copy(poolb, pool.at[pl.ds(s, 1)])
    plsc.subcore_barrier()

    @pl.when(jnp.logical_and(c == 0, s == 0))
    def _mlp():
        pltpu.sync_copy(pool, poolall)
        lo = poolall[0, pl.ds(0, 16)]
        hi = poolall[0, pl.ds(16, 16)]
        for r in range(1, N_TILES):
            lo = lo + poolall[r, pl.ds(0, 16)]
            hi = hi + poolall[r, pl.ds(16, 16)]
        inv_n = jnp.float32(1.0 / N_NODES)
        pooled = [lo * inv_n, hi * inv_n]
        h = [wbf1[pl.ds(16 * j, 16)] for j in range(4)]
        for k in range(F2):
            a = pooled[k // 16][k % 16]
            for j in range(4):
                h[j] = h[j] + a * wWf1[k, pl.ds(16 * j, 16)]
        h = [jnp.maximum(hj, 0.0) for hj in h]
        o = wbf2[...]
        for k in range(64):
            o = o + h[k // 16][k % 16] * wWf2[k, :]
        outb[0, :] = o
        pltpu.sync_copy(outb, out_ref)


def _make_sc_kernel(GT):
    mesh = plsc.VectorSubcoreMesh(core_axis_name="c", subcore_axis_name="s")
    f32 = jnp.float32
    return pl.kernel(
        functools.partial(_sc_body, GT),
        out_type=jax.ShapeDtypeStruct((1, 16), f32),
        mesh=mesh,
        compiler_params=pltpu.CompilerParams(use_tc_tiling_on_sc=False),
        scratch_types=[
            pltpu.VMEM_SHARED((NR, F1), f32),
            pltpu.VMEM_SHARED((NR, F1), f32),
            pltpu.VMEM_SHARED((NR,), f32),
            pltpu.VMEM_SHARED((N_TILES, F2), f32),
            pltpu.VMEM((GT, G), jnp.int32),
            pltpu.VMEM((GT, G), jnp.int32),
            pltpu.VMEM((G, F1), f32),
            pltpu.VMEM((NPT, F1), f32),
            pltpu.VMEM((NPT, F1), f32),
            pltpu.VMEM((NPT,), f32),
            pltpu.VMEM((G,), f32),
            pltpu.VMEM((F1,), f32),
            pltpu.VMEM((F1, F2), f32),
            pltpu.VMEM((F2,), f32),
            pltpu.VMEM((F2, 64), f32),
            pltpu.VMEM((64,), f32),
            pltpu.VMEM((64, 16), f32),
            pltpu.VMEM((16,), f32),
            pltpu.VMEM((1, F2), f32),
            pltpu.VMEM((N_TILES, F2), f32),
            pltpu.VMEM((1, 16), f32),
        ],
    )


def kernel(x, edge_index, W1, b1, W2, b2, Wf1, bf1, Wf2, bf2):
    n = x.shape[0]
    e = edge_index.shape[1]
    ei = edge_index.astype(jnp.int32)
    gt = (e + N_TILES * G - 1) // (N_TILES * G)
    gt = ((gt + 7) // 8) * 8
    ep = gt * N_TILES * G
    pad = ep - e
    pad_idx = (jnp.arange(pad, dtype=jnp.int32) % (NR - N_NODES)) + N_NODES
    src2 = jnp.concatenate([ei[0], pad_idx]).reshape(ep // G, G)
    dst2 = jnp.concatenate([ei[1], pad_idx]).reshape(ep // G, G)

    xw = pl.pallas_call(
        _mm_body,
        out_shape=jax.ShapeDtypeStruct((n, F1), jnp.float32),
    )(x, W1)
    xw_pad = jnp.pad(xw, ((0, NR - n), (0, 0)))

    wf2p = jnp.pad(Wf2, ((0, 0), (0, 16 - Wf2.shape[1])))
    bf2p = jnp.pad(bf2, (0, 16 - bf2.shape[0]))

    out16 = _make_sc_kernel(gt)(
        xw_pad, src2, dst2, b1, W2, b2, Wf1, bf1, wf2p, bf2p)
    return out16[:, :Wf2.shape[1]]


# speedup vs baseline: 69.1948x; 1.4155x over previous
import functools

import jax
import jax.numpy as jnp
from jax import lax
from jax.experimental import pallas as pl
from jax.experimental.pallas import tpu as pltpu
from jax.experimental.pallas import tpu_sc as plsc

N_NODES = 10000
NR = 10240
NPT = NR // 16
F1 = 16
F2 = 32
G = 128
NBUF = 8
N_TILES = 16


def _mm_body(x_ref, w_ref, o_ref):
    o_ref[...] = jnp.dot(x_ref[...], w_ref[...],
                         preferred_element_type=jnp.float32)


def _rsqrt16(d):
    i = lax.bitcast_convert_type(d, jnp.int32)
    y = lax.bitcast_convert_type(jnp.int32(0x5F3759DF) - (i >> 1), jnp.float32)
    for _ in range(3):
        y = y * (1.5 - 0.5 * d * y * y)
    return y


def _sc_body(GT, xw_ref, src_ref, dst_ref, b1_ref, w2_ref, b2_ref,
             wf1_ref, bf1_ref, wf2_ref, bf2_ref, out_ref,
             table, acc, deg, pool,
             sidx, didx, rowbuf, nbin, nbout, dv, ones,
             wb1, wW2, wb2, wWf1, wbf1, wWf2, wbf2,
             poolb, poolall, outb, gsem, ssem, dmasem):
    c = lax.axis_index("c")
    s = lax.axis_index("s")
    nbase = s * NPT
    gbase = s * GT

    pltpu.sync_copy(src_ref.at[pl.ds(gbase, GT)], sidx)
    pltpu.sync_copy(dst_ref.at[pl.ds(gbase, GT)], didx)
    pltpu.sync_copy(b1_ref, wb1)
    pltpu.sync_copy(w2_ref, wW2)
    pltpu.sync_copy(b2_ref, wb2)
    pltpu.sync_copy(wf1_ref, wWf1)
    pltpu.sync_copy(bf1_ref, wbf1)
    pltpu.sync_copy(wf2_ref, wWf2)
    pltpu.sync_copy(bf2_ref, wbf2)
    for j in range(G // 16):
        ones[pl.ds(j * 16, 16)] = jnp.full((16,), 1.0, jnp.float32)

    def initdeg(j, carry):
        dv[pl.ds(j * 16, 16)] = jnp.full((16,), 1.0, jnp.float32)
        return carry
    lax.fori_loop(0, NPT // 16, initdeg, 0)
    pltpu.sync_copy(dv, deg.at[pl.ds(nbase, NPT)])
    plsc.subcore_barrier()

    def dpass(b, carry):
        ds = [pltpu.async_copy(ones, deg.at[didx.at[b * NBUF + j]],
                               dmasem.at[j], add=True)
              for j in range(NBUF)]
        for d in ds:
            d.wait()
        return carry
    lax.fori_loop(0, GT // NBUF, dpass, 0)
    plsc.subcore_barrier()

    pltpu.sync_copy(deg.at[pl.ds(nbase, NPT)], dv)

    def newton(j, carry):
        dv[pl.ds(j * 16, 16)] = _rsqrt16(dv[pl.ds(j * 16, 16)])
        return carry
    lax.fori_loop(0, NPT // 16, newton, 0)
    pltpu.sync_copy(xw_ref.at[pl.ds(nbase, NPT)], nbin)

    def scale1(j, carry):
        dvec = dv[pl.ds(j * 16, 16)]
        for l in range(16):
            i = j * 16 + l
            nbout[i, :] = nbin[i, :] * dvec[l]
        return carry
    lax.fori_loop(0, NPT // 16, scale1, 0)
    pltpu.sync_copy(nbout, table.at[pl.ds(nbase, NPT)])
    pltpu.sync_copy(nbout, acc.at[pl.ds(nbase, NPT)])
    plsc.subcore_barrier()

    def epass(b, carry):
        gb = b * NBUF
        gds = [pltpu.async_copy(table.at[sidx.at[gb + j]], rowbuf.at[j],
                                gsem.at[j])
               for j in range(NBUF)]
        sds = []
        for j in range(NBUF):
            gds[j].wait()
            sds.append(pltpu.async_copy(rowbuf.at[j],
                                        acc.at[didx.at[gb + j]],
                                        ssem.at[j], add=True))
        for d in sds:
            d.wait()
        return carry

    lax.fori_loop(0, GT // NBUF, epass, 0)
    plsc.subcore_barrier()

    pltpu.sync_copy(acc.at[pl.ds(nbase, NPT)], nbin)
    b1v = wb1[...]

    def stage_c(j, carry):
        dvec = dv[pl.ds(j * 16, 16)]
        for l in range(16):
            i = j * 16 + l
            d = dvec[l]
            h = jnp.maximum(nbin[i, :] * d + b1v, 0.0)
            nbout[i, :] = h * d
        return carry
    lax.fori_loop(0, NPT // 16, stage_c, 0)
    pltpu.sync_copy(nbout, table.at[pl.ds(nbase, NPT)])
    pltpu.sync_copy(nbout, acc.at[pl.ds(nbase, NPT)])
    plsc.subcore_barrier()

    lax.fori_loop(0, GT // NBUF, epass, 0)
    plsc.subcore_barrier()

    pltpu.sync_copy(acc.at[pl.ds(nbase, NPT)], nbin)
    b2lo = wb2[pl.ds(0, 16)]
    b2hi = wb2[pl.ds(16, 16)]

    def stage_e(j, carry):
        plo, phi = carry
        dvec = dv[pl.ds(j * 16, 16)]
        for l in range(16):
            i = j * 16 + l
            aggv = nbin[i, :] * dvec[l]
            lo = b2lo
            hi = b2hi
            for k in range(F1):
                a = aggv[k]
                lo = lo + a * wW2[k, pl.ds(0, 16)]
                hi = hi + a * wW2[k, pl.ds(16, 16)]
            lo = jnp.maximum(lo, 0.0)
            hi = jnp.maximum(hi, 0.0)
            m = jnp.where(nbase + i < N_NODES, 1.0, 0.0)
            plo = plo + lo * m
            phi = phi + hi * m
        return (plo, phi)

    zero16 = jnp.zeros((16,), jnp.float32)
    plo, phi = lax.fori_loop(0, NPT // 16, stage_e, (zero16, zero16))
    poolb[0, pl.ds(0, 16)] = plo
    poolb[0, pl.ds(16, 16)] = phi
    pltpu.sync_copy(poolb, pool.at[pl.ds(s, 1)])
    plsc.subcore_barrier()

    @pl.when(jnp.logical_and(c == 0, s == 0))
    def _mlp():
        pltpu.sync_copy(pool, poolall)
        lo = poolall[0, pl.ds(0, 16)]
        hi = poolall[0, pl.ds(16, 16)]
        for r in range(1, N_TILES):
            lo = lo + poolall[r, pl.ds(0, 16)]
            hi = hi + poolall[r, pl.ds(16, 16)]
        inv_n = jnp.float32(1.0 / N_NODES)
        pooled = [lo * inv_n, hi * inv_n]
        h = [wbf1[pl.ds(16 * j, 16)] for j in range(4)]
        for k in range(F2):
            a = pooled[k // 16][k % 16]
            for j in range(4):
                h[j] = h[j] + a * wWf1[k, pl.ds(16 * j, 16)]
        h = [jnp.maximum(hj, 0.0) for hj in h]
        o = wbf2[...]
        for k in range(64):
            o = o + h[k // 16][k % 16] * wWf2[k, :]
        outb[0, :] = o
        pltpu.sync_copy(outb, out_ref)


def _make_sc_kernel(GT):
    mesh = plsc.VectorSubcoreMesh(core_axis_name="c", subcore_axis_name="s")
    f32 = jnp.float32
    return pl.kernel(
        functools.partial(_sc_body, GT),
        out_type=jax.ShapeDtypeStruct((1, 16), f32),
        mesh=mesh,
        compiler_params=pltpu.CompilerParams(use_tc_tiling_on_sc=False),
        scratch_types=[
            pltpu.VMEM_SHARED((NR, F1), f32),
            pltpu.VMEM_SHARED((NR, F1), f32),
            pltpu.VMEM_SHARED((NR,), f32),
            pltpu.VMEM_SHARED((N_TILES, F2), f32),
            pltpu.VMEM((GT, G), jnp.int32),
            pltpu.VMEM((GT, G), jnp.int32),
            pltpu.VMEM((NBUF, G, F1), f32),
            pltpu.VMEM((NPT, F1), f32),
            pltpu.VMEM((NPT, F1), f32),
            pltpu.VMEM((NPT,), f32),
            pltpu.VMEM((G,), f32),
            pltpu.VMEM((F1,), f32),
            pltpu.VMEM((F1, F2), f32),
            pltpu.VMEM((F2,), f32),
            pltpu.VMEM((F2, 64), f32),
            pltpu.VMEM((64,), f32),
            pltpu.VMEM((64, 16), f32),
            pltpu.VMEM((16,), f32),
            pltpu.VMEM((1, F2), f32),
            pltpu.VMEM((N_TILES, F2), f32),
            pltpu.VMEM((1, 16), f32),
            pltpu.SemaphoreType.DMA((NBUF,)),
            pltpu.SemaphoreType.DMA((NBUF,)),
            pltpu.SemaphoreType.DMA((NBUF,)),
        ],
    )


def kernel(x, edge_index, W1, b1, W2, b2, Wf1, bf1, Wf2, bf2):
    n = x.shape[0]
    e = edge_index.shape[1]
    ei = edge_index.astype(jnp.int32)
    gt = (e + N_TILES * G - 1) // (N_TILES * G)
    gt = ((gt + 7) // 8) * 8
    ep = gt * N_TILES * G
    pad = ep - e
    pad_idx = (jnp.arange(pad, dtype=jnp.int32) % (NR - N_NODES)) + N_NODES
    src2 = jnp.concatenate([ei[0], pad_idx]).reshape(ep // G, G)
    dst2 = jnp.concatenate([ei[1], pad_idx]).reshape(ep // G, G)

    xw = pl.pallas_call(
        _mm_body,
        out_shape=jax.ShapeDtypeStruct((n, F1), jnp.float32),
    )(x, W1)
    xw_pad = jnp.pad(xw, ((0, NR - n), (0, 0)))

    wf2p = jnp.pad(Wf2, ((0, 0), (0, 16 - Wf2.shape[1])))
    bf2p = jnp.pad(bf2, (0, 16 - bf2.shape[0]))

    out16 = _make_sc_kernel(gt)(
        xw_pad, src2, dst2, b1, W2, b2, Wf1, bf1, wf2p, bf2p)
    return out16[:, :Wf2.shape[1]]
